# Initial kernel scaffold; baseline (speedup 1.0000x reference)
import jax
import jax.numpy as jnp
from jax.experimental import pallas as pl


def kernel(coords_world, lifted_features):
    raise NotImplementedError("write your pallas kernel here")


# speedup vs baseline: 1.8367x; 1.8367x over previous
import functools

import jax
import jax.numpy as jnp
from jax import lax
from jax.experimental import pallas as pl
from jax.experimental.pallas import tpu as pltpu
from jax.experimental.pallas import tpu_sc as plsc

X_MIN, X_MAX = -50.0, 50.0
Y_MIN, Y_MAX = -50.0, 50.0
BEV_W = 200
BEV_H = 200
BEV = BEV_W * BEV_H
TRASH = BEV
ACC = 40016

NC, NS, L = 2, 16, 16
NW = NC * NS

B = 2
ND = 354
HW = 704
C = 64
SLABS_PER_TILE = (ND + NW - 1) // NW
CHUNK = 6
NCHUNK = ND // CHUNK
VECS = HW // L

_mesh = plsc.VectorSubcoreMesh(
    core_axis_name="c", subcore_axis_name="s", num_cores=NC, num_subcores=NS
)


def _worker_id():
    return lax.axis_index("s") * NC + lax.axis_index("c")


def _phase1_body(x_hbm, y_hbm, idx_hbm, xbuf, ybuf, ibuf):
    w = _worker_id()
    for b in range(B):
        @pl.loop(0, SLABS_PER_TILE)
        def _slab(k):
            s = w + k * NW
            @pl.when(s < ND)
            def _():
                pltpu.sync_copy(x_hbm.at[b, s], xbuf)
                pltpu.sync_copy(y_hbm.at[b, s], ybuf)

                @pl.loop(0, VECS)
                def _vec(j):
                    x = xbuf[pl.ds(j * L, L)]
                    y = ybuf[pl.ds(j * L, L)]
                    xf = (x - X_MIN) * (BEV_W / (X_MAX - X_MIN))
                    yf = (y - Y_MIN) * (BEV_H / (Y_MAX - Y_MIN))
                    xf = jnp.minimum(jnp.maximum(xf, -2.0e9), 2.0e9)
                    yf = jnp.minimum(jnp.maximum(yf, -2.0e9), 2.0e9)
                    xi = xf.astype(jnp.int32)
                    yi = yf.astype(jnp.int32)
                    valid = (
                        (xi >= 0) & (xi < BEV_W) & (yi >= 0) & (yi < BEV_H)
                    )
                    lin = yi * BEV_W + xi
                    lin = jnp.where(valid, lin, TRASH)
                    ibuf[pl.ds(j * L, L)] = lin

                pltpu.sync_copy(ibuf, idx_hbm.at[b, s])


_phase1 = pl.kernel(
    _phase1_body,
    out_type=jax.ShapeDtypeStruct((B, ND, HW), jnp.int32),
    mesh=_mesh,
    compiler_params=pltpu.CompilerParams(
        use_tc_tiling_on_sc=False, needs_layout_passes=False
    ),
    scratch_types=[
        pltpu.VMEM((HW,), jnp.float32),
        pltpu.VMEM((HW,), jnp.float32),
        pltpu.VMEM((HW,), jnp.int32),
    ],
)


def _phase2_body(idx_hbm, feats_hbm, out_hbm, acc0, acc1, ibuf, f0buf, f1buf):
    w = _worker_id()
    c0 = w * 2
    c1 = c0 + 1
    zeros = jnp.zeros((L,), jnp.float32)
    for b in range(B):
        @pl.loop(0, ACC // L, unroll=8)
        def _zero(i):
            acc0[pl.ds(i * L, L)] = zeros
            acc1[pl.ds(i * L, L)] = zeros

        @pl.loop(0, NCHUNK)
        def _chunk(k):
            s0 = k * CHUNK
            pltpu.sync_copy(idx_hbm.at[b, pl.ds(s0, CHUNK)], ibuf)
            pltpu.sync_copy(feats_hbm.at[b, pl.ds(s0, CHUNK), c0], f0buf)
            pltpu.sync_copy(feats_hbm.at[b, pl.ds(s0, CHUNK), c1], f1buf)

            @pl.loop(0, CHUNK)
            def _slab(i):
                @pl.loop(0, VECS, unroll=4)
                def _vec(j):
                    iv = ibuf[i, pl.ds(j * L, L)]
                    v0 = f0buf[i, pl.ds(j * L, L)]
                    plsc.addupdate_scatter(acc0, [iv], v0)
                    v1 = f1buf[i, pl.ds(j * L, L)]
                    plsc.addupdate_scatter(acc1, [iv], v1)

        pltpu.sync_copy(acc0.at[pl.ds(0, BEV)], out_hbm.at[b, c0])
        pltpu.sync_copy(acc1.at[pl.ds(0, BEV)], out_hbm.at[b, c1])


_phase2 = pl.kernel(
    _phase2_body,
    out_type=jax.ShapeDtypeStruct((B, C, BEV), jnp.float32),
    mesh=_mesh,
    compiler_params=pltpu.CompilerParams(
        use_tc_tiling_on_sc=False, needs_layout_passes=False
    ),
    scratch_types=[
        pltpu.VMEM((ACC,), jnp.float32),
        pltpu.VMEM((ACC,), jnp.float32),
        pltpu.VMEM((CHUNK, HW), jnp.int32),
        pltpu.VMEM((CHUNK, HW), jnp.float32),
        pltpu.VMEM((CHUNK, HW), jnp.float32),
    ],
)


def kernel(coords_world, lifted_features):
    b, n, d, c, h, w = lifted_features.shape
    xs = coords_world[..., 0].reshape(b, n * d, h * w)
    ys = coords_world[..., 1].reshape(b, n * d, h * w)
    feats = lifted_features.reshape(b, n * d, c, h * w)
    idx = _phase1(xs, ys)
    bev = _phase2(idx, feats)
    return bev.reshape(b, c, BEV_H, BEV_W)


# speedup vs baseline: 2.1281x; 1.1587x over previous
import jax
import jax.numpy as jnp
from jax import lax
from jax.experimental import pallas as pl
from jax.experimental.pallas import tpu as pltpu
from jax.experimental.pallas import tpu_sc as plsc

X_MIN, X_MAX = -50.0, 50.0
Y_MIN, Y_MAX = -50.0, 50.0
BEV_W = 200
BEV_H = 200
BEV = BEV_W * BEV_H
TRASH = BEV
ACC = 40016

NC, NS, L = 2, 16, 16
NW = NC * NS

B = 2
ND = 354
HW = 704
C = 64
SLABS_PER_TILE = (ND + NW - 1) // NW
CHUNK = 6
NCHUNK = ND // CHUNK
VECS = HW // L

_mesh = plsc.VectorSubcoreMesh(
    core_axis_name="c", subcore_axis_name="s", num_cores=NC, num_subcores=NS
)
_params = pltpu.CompilerParams(
    use_tc_tiling_on_sc=False, needs_layout_passes=False
)


def _worker_id():
    return lax.axis_index("s") * NC + lax.axis_index("c")


def _phase1_body(coords_hbm, idx_hbm, cbuf, ibuf):
    w = _worker_id()
    lane = lax.iota(jnp.int32, L)
    for b in range(B):
        @pl.loop(0, SLABS_PER_TILE)
        def _slab(k):
            s = w + k * NW
            @pl.when(s < ND)
            def _():
                pltpu.sync_copy(coords_hbm.at[b, s], cbuf)

                @pl.loop(0, VECS)
                def _vec(j):
                    g = j * (3 * L) + lane * 3
                    x = plsc.load_gather(cbuf, [g])
                    y = plsc.load_gather(cbuf, [g + 1])
                    xf = (x - X_MIN) * (BEV_W / (X_MAX - X_MIN))
                    yf = (y - Y_MIN) * (BEV_H / (Y_MAX - Y_MIN))
                    xf = jnp.minimum(jnp.maximum(xf, -2.0e9), 2.0e9)
                    yf = jnp.minimum(jnp.maximum(yf, -2.0e9), 2.0e9)
                    xi = xf.astype(jnp.int32)
                    yi = yf.astype(jnp.int32)
                    valid = (
                        (xi >= 0) & (xi < BEV_W) & (yi >= 0) & (yi < BEV_H)
                    )
                    lin = yi * BEV_W + xi
                    lin = jnp.where(valid, lin, TRASH)
                    ibuf[pl.ds(j * L, L)] = lin

                pltpu.sync_copy(ibuf, idx_hbm.at[b, s])


_phase1 = pl.kernel(
    _phase1_body,
    out_type=jax.ShapeDtypeStruct((B, ND, HW), jnp.int32),
    mesh=_mesh,
    compiler_params=_params,
    scratch_types=[
        pltpu.VMEM((3 * HW,), jnp.float32),
        pltpu.VMEM((HW,), jnp.int32),
    ],
)


def _phase2_body(idx_hbm, feats_hbm, out_hbm, acc0, acc1, ibuf, f0buf, f1buf, sems):
    w = _worker_id()
    c0 = w * 2
    c1 = c0 + 1
    zeros = jnp.zeros((L,), jnp.float32)
    for b in range(B):
        def copies(slot, k):
            s0 = k * CHUNK
            return (
                pltpu.make_async_copy(
                    idx_hbm.at[b, pl.ds(s0, CHUNK)], ibuf.at[slot], sems.at[slot]
                ),
                pltpu.make_async_copy(
                    feats_hbm.at[b, pl.ds(s0, CHUNK), c0], f0buf.at[slot],
                    sems.at[slot],
                ),
                pltpu.make_async_copy(
                    feats_hbm.at[b, pl.ds(s0, CHUNK), c1], f1buf.at[slot],
                    sems.at[slot],
                ),
            )

        def issue(slot, k):
            for cp in copies(slot, k):
                cp.start()

        def drain(slot, k):
            for cp in copies(slot, k):
                cp.wait()

        @pl.loop(0, ACC // L, unroll=8)
        def _zero(i):
            acc0[pl.ds(i * L, L)] = zeros
            acc1[pl.ds(i * L, L)] = zeros

        issue(0, 0)

        @pl.loop(0, NCHUNK)
        def _chunk(k):
            slot = k & 1
            @pl.when(k + 1 < NCHUNK)
            def _():
                issue(1 - slot, k + 1)

            drain(slot, k)

            @pl.loop(0, CHUNK)
            def _slab(i):
                @pl.loop(0, VECS, unroll=4)
                def _vec(j):
                    iv = ibuf[slot, i, pl.ds(j * L, L)]
                    v0 = f0buf[slot, i, pl.ds(j * L, L)]
                    plsc.addupdate_scatter(acc0, [iv], v0)
                    v1 = f1buf[slot, i, pl.ds(j * L, L)]
                    plsc.addupdate_scatter(acc1, [iv], v1)

        pltpu.sync_copy(acc0.at[pl.ds(0, BEV)], out_hbm.at[b, c0])
        pltpu.sync_copy(acc1.at[pl.ds(0, BEV)], out_hbm.at[b, c1])


_phase2 = pl.kernel(
    _phase2_body,
    out_type=jax.ShapeDtypeStruct((B, C, BEV), jnp.float32),
    mesh=_mesh,
    compiler_params=_params,
    scratch_types=[
        pltpu.VMEM((ACC,), jnp.float32),
        pltpu.VMEM((ACC,), jnp.float32),
        pltpu.VMEM((2, CHUNK, HW), jnp.int32),
        pltpu.VMEM((2, CHUNK, HW), jnp.float32),
        pltpu.VMEM((2, CHUNK, HW), jnp.float32),
        pltpu.SemaphoreType.DMA((2,)),
    ],
)


def kernel(coords_world, lifted_features):
    b, n, d, c, h, w = lifted_features.shape
    coords = coords_world.reshape(b, n * d, h * w * 3)
    feats = lifted_features.reshape(b, n * d, c, h * w)
    idx = _phase1(coords)
    bev = _phase2(idx, feats)
    return bev.reshape(b, c, BEV_H, BEV_W)
